# baseline (speedup 1.0000x reference)
import functools

import jax
import jax.numpy as jnp
from jax import lax
from jax.experimental import pallas as pl
from jax.experimental.pallas import tpu as pltpu
from jax.experimental.pallas import tpu_sc as plsc

_TOTAL = 16384
_EMB = 128
_BATCH = 16
_NC = 2
_NS = 16
_NW = _NC * _NS
_B_PER_W = _TOTAL // _NW


@functools.cache
def _build_gather():
    mesh = plsc.VectorSubcoreMesh(core_axis_name="c", subcore_axis_name="s")

    @functools.partial(
        pl.kernel,
        mesh=mesh,
        out_type=(
            jax.ShapeDtypeStruct((_TOTAL, _EMB), jnp.float32),
            jax.ShapeDtypeStruct((_BATCH + 1,), jnp.int32),
        ),
        scratch_types=[
            pltpu.VMEM((_B_PER_W,), jnp.int32),
            pltpu.VMEM((_B_PER_W, _EMB), jnp.float32),
            pltpu.VMEM((_BATCH + 1,), jnp.int32),
            pltpu.SemaphoreType.DMA,
        ],
    )
    def gather(table_hbm, idx_hbm, cu_hbm, out_hbm, bnd_hbm, idx_v, rows_v,
               cu_v, sem):
        wid = lax.axis_index("s") * _NC + lax.axis_index("c")
        base = wid * _B_PER_W
        pltpu.sync_copy(idx_hbm.at[pl.ds(base, _B_PER_W)], idx_v)
        pltpu.async_copy(table_hbm.at[idx_v], rows_v, sem).wait()

        @pl.when(wid == 0)
        def _copy_boundaries():
            pltpu.sync_copy(cu_hbm, cu_v)
            pltpu.sync_copy(cu_v, bnd_hbm)

        pltpu.sync_copy(rows_v, out_hbm.at[pl.ds(base, _B_PER_W)])

    return gather


def kernel(token_ids, cu_seqlens, table):
    return _build_gather()(table, token_ids.astype(jnp.int32),
                           cu_seqlens.astype(jnp.int32))


# speedup vs baseline: 1.0095x; 1.0095x over previous
import functools

import jax
import jax.numpy as jnp
from jax import lax
from jax.experimental import pallas as pl
from jax.experimental.pallas import tpu as pltpu
from jax.experimental.pallas import tpu_sc as plsc

_TOTAL = 16384
_EMB = 128
_BATCH = 16
_NC = 2
_NS = 16
_NW = _NC * _NS
_B_PER_W = _TOTAL // _NW


@functools.cache
def _build_gather():
    mesh = plsc.VectorSubcoreMesh(core_axis_name="c", subcore_axis_name="s")

    @functools.partial(
        pl.kernel,
        mesh=mesh,
        out_type=jax.ShapeDtypeStruct((_TOTAL, _EMB), jnp.float32),
        scratch_types=[
            pltpu.VMEM((_B_PER_W,), jnp.int32),
            pltpu.VMEM((_B_PER_W, _EMB), jnp.float32),
        ],
    )
    def gather(table_hbm, idx_hbm, out_hbm, idx_v, rows_v):
        wid = lax.axis_index("s") * _NC + lax.axis_index("c")
        base = wid * _B_PER_W
        pltpu.sync_copy(idx_hbm.at[pl.ds(base, _B_PER_W)], idx_v)
        pltpu.sync_copy(table_hbm.at[idx_v], rows_v)
        pltpu.sync_copy(rows_v, out_hbm.at[pl.ds(base, _B_PER_W)])

    return gather


def kernel(token_ids, cu_seqlens, table):
    all_embs = _build_gather()(table, token_ids.astype(jnp.int32))
    return (all_embs, cu_seqlens)


# speedup vs baseline: 1.0106x; 1.0010x over previous
import functools

import jax
import jax.numpy as jnp
from jax import lax
from jax.experimental import pallas as pl
from jax.experimental.pallas import tpu as pltpu
from jax.experimental.pallas import tpu_sc as plsc

_NC = 2
_NS = 16
_NW = _NC * _NS


@functools.cache
def _build_gather(total, emb):
    rows_per_tile = total // _NW
    mesh = plsc.VectorSubcoreMesh(core_axis_name="c", subcore_axis_name="s")

    @functools.partial(
        pl.kernel,
        mesh=mesh,
        out_type=jax.ShapeDtypeStruct((total, emb), jnp.float32),
        scratch_types=[
            pltpu.VMEM((rows_per_tile,), jnp.int32),
            pltpu.VMEM((rows_per_tile, emb), jnp.float32),
        ],
    )
    def gather(table_hbm, idx_hbm, out_hbm, idx_v, rows_v):
        wid = lax.axis_index("s") * _NC + lax.axis_index("c")
        base = wid * rows_per_tile
        pltpu.sync_copy(idx_hbm.at[pl.ds(base, rows_per_tile)], idx_v)
        pltpu.sync_copy(table_hbm.at[idx_v], rows_v)
        pltpu.sync_copy(rows_v, out_hbm.at[pl.ds(base, rows_per_tile)])

    return gather


def kernel(token_ids, cu_seqlens, table):
    total = token_ids.shape[0]
    emb = table.shape[1]
    all_embs = _build_gather(total, emb)(table, token_ids.astype(jnp.int32))
    return (all_embs, cu_seqlens)
